# Initial kernel scaffold; baseline (speedup 1.0000x reference)
import jax
import jax.numpy as jnp
from jax.experimental import pallas as pl


def kernel(user, item, embed_user, embed_item, W1, b1, W2, b2, Wp, bp):
    raise NotImplementedError("write your pallas kernel here")


# speedup vs baseline: 8.2681x; 8.2681x over previous
import functools

import jax
import jax.numpy as jnp
from jax import lax
from jax.experimental import pallas as pl
from jax.experimental.pallas import tpu as pltpu
from jax.experimental.pallas import tpu_sc as plsc

B = 16384
D = 128
NC = 2
NS = 16
NW = NC * NS
PER_W = B // NW
CHUNK = 128
CHUNKS = PER_W // CHUNK


SEG = 256
SEG_CHUNKS = SEG // CHUNK
NSEG = 2 * PER_W // SEG


def _gather_body(uidx_hbm, iidx_hbm, utab_hbm, itab_hbm, out_u, out_i,
                 uidx_v, iidx_v, rows_a, rows_b, gsem_a, gsem_b,
                 osem_a, osem_b):
    wid = lax.axis_index("s") * NC + lax.axis_index("c")
    base = wid * PER_W
    pltpu.sync_copy(uidx_hbm.at[wid], uidx_v)
    pltpu.sync_copy(iidx_hbm.at[wid], iidx_v)
    bufs = [rows_a, rows_b]
    gsems = [gsem_a, gsem_b]
    osems = [osem_a, osem_b]
    segs = [(uidx_v, 0, out_u, base), (uidx_v, SEG_CHUNKS, out_u, base + SEG),
            (iidx_v, 0, out_i, base), (iidx_v, SEG_CHUNKS, out_i, base + SEG)]
    gh = [None] * NSEG
    oh = [None] * NSEG
    for k in range(NSEG + 1):
        if k < NSEG:
            if k >= 2:
                oh[k - 2].wait()
            idxv, coff, _, _ = segs[k]
            b = k % 2
            gh[k] = [pltpu.async_copy(
                (utab_hbm if idxv is uidx_v else itab_hbm).at[idxv.at[coff + j]],
                bufs[b].at[pl.ds(j * CHUNK, CHUNK)], gsems[b])
                for j in range(SEG_CHUNKS)]
        if k >= 1:
            p = k - 1
            for h in gh[p]:
                h.wait()
            _, _, outref, roff = segs[p]
            oh[p] = pltpu.async_copy(bufs[p % 2], outref.at[pl.ds(roff, SEG)],
                                     osems[p % 2])
    oh[NSEG - 2].wait()
    oh[NSEG - 1].wait()


@functools.partial(jax.jit, static_argnums=())
def _gather(uidx, iidx, utab, itab):
    mesh = plsc.VectorSubcoreMesh(core_axis_name="c", subcore_axis_name="s")
    k = functools.partial(
        pl.kernel,
        mesh=mesh,
        out_type=[jax.ShapeDtypeStruct((B, D), jnp.float32),
                  jax.ShapeDtypeStruct((B, D), jnp.float32)],
        scratch_types=[
            pltpu.VMEM((CHUNKS, CHUNK), jnp.int32),
            pltpu.VMEM((CHUNKS, CHUNK), jnp.int32),
            pltpu.VMEM((SEG, D), jnp.float32),
            pltpu.VMEM((SEG, D), jnp.float32),
            pltpu.SemaphoreType.DMA,
            pltpu.SemaphoreType.DMA,
            pltpu.SemaphoreType.DMA,
            pltpu.SemaphoreType.DMA,
        ],
    )(_gather_body)
    return k(uidx, iidx, utab, itab)


def _mlp_body(eu, ei, w1a, w1b, b1, w2, b2, wp, bp, out):
    x = (jnp.dot(eu[...], w1a[...], preferred_element_type=jnp.float32)
         + jnp.dot(ei[...], w1b[...], preferred_element_type=jnp.float32)
         + b1[...])
    h = jnp.maximum(x, 0.0)
    h2 = jnp.maximum(
        jnp.dot(h, w2[...], preferred_element_type=jnp.float32) + b2[...], 0.0)
    out[...] = jnp.sum(h2 * wp[...], axis=1) + bp[0, 0]


def _mlp(eu, ei, w1a, w1b, b1, w2, b2, wp, bp):
    BLK = 2048
    grid = (B // BLK,)
    full = lambda i: (0, 0)
    return pl.pallas_call(
        _mlp_body,
        grid=grid,
        in_specs=[
            pl.BlockSpec((BLK, D), lambda i: (i, 0)),
            pl.BlockSpec((BLK, D), lambda i: (i, 0)),
            pl.BlockSpec((D, 64), full),
            pl.BlockSpec((D, 64), full),
            pl.BlockSpec((1, 64), full),
            pl.BlockSpec((64, 16), full),
            pl.BlockSpec((1, 16), full),
            pl.BlockSpec((1, 16), full),
            pl.BlockSpec((1, 1), full),
        ],
        out_specs=pl.BlockSpec((BLK,), lambda i: (i,)),
        out_shape=jax.ShapeDtypeStruct((B,), jnp.float32),
    )(eu, ei, w1a, w1b, b1, w2, b2, wp, bp)


def kernel(user, item, embed_user, embed_item, W1, b1, W2, b2, Wp, bp):
    uidx = user.astype(jnp.int32).reshape(NW, CHUNKS, CHUNK)
    iidx = item.astype(jnp.int32).reshape(NW, CHUNKS, CHUNK)
    eu, ei = _gather(uidx, iidx, embed_user, embed_item)
    w1a = W1[:D]
    w1b = W1[D:]
    pred = _mlp(eu, ei, w1a, w1b,
                b1.reshape(1, 64), W2, b2.reshape(1, 16),
                Wp.reshape(1, 16), bp.reshape(1, 1))
    return pred


# speedup vs baseline: 9.5298x; 1.1526x over previous
import functools

import jax
import jax.numpy as jnp
from jax import lax
from jax.experimental import pallas as pl
from jax.experimental.pallas import tpu as pltpu
from jax.experimental.pallas import tpu_sc as plsc

B = 16384
D = 128
NC = 2
NS = 16
NW = NC * NS
PER_W = B // NW
CHUNK = 128
CHUNKS = PER_W // CHUNK


SEG = 256
SEG_CHUNKS = SEG // CHUNK
NSEG = 2 * PER_W // SEG


def _gather_body(uidx_hbm, iidx_hbm, utab_hbm, itab_hbm, out_u, out_i,
                 uidx_v, iidx_v, rows_a, rows_b, gsem_a, gsem_b,
                 osem_a, osem_b):
    wid = lax.axis_index("s") * NC + lax.axis_index("c")
    base = wid * PER_W
    pltpu.sync_copy(uidx_hbm.at[wid], uidx_v)
    pltpu.sync_copy(iidx_hbm.at[wid], iidx_v)
    bufs = [rows_a, rows_b]
    gsems = [gsem_a, gsem_b]
    osems = [osem_a, osem_b]
    segs = [(uidx_v, 0, out_u, base), (uidx_v, SEG_CHUNKS, out_u, base + SEG),
            (iidx_v, 0, out_i, base), (iidx_v, SEG_CHUNKS, out_i, base + SEG)]
    gh = [None] * NSEG
    oh = [None] * NSEG
    for k in range(NSEG + 1):
        if k < NSEG:
            if k >= 2:
                oh[k - 2].wait()
            idxv, coff, _, _ = segs[k]
            b = k % 2
            gh[k] = [pltpu.async_copy(
                (utab_hbm if idxv is uidx_v else itab_hbm).at[idxv.at[coff + j]],
                bufs[b].at[pl.ds(j * CHUNK, CHUNK)], gsems[b])
                for j in range(SEG_CHUNKS)]
        if k >= 1:
            p = k - 1
            for h in gh[p]:
                h.wait()
            _, _, outref, roff = segs[p]
            oh[p] = pltpu.async_copy(bufs[p % 2], outref.at[pl.ds(roff, SEG)],
                                     osems[p % 2])
    oh[NSEG - 2].wait()
    oh[NSEG - 1].wait()


@functools.partial(jax.jit, static_argnums=())
def _gather(uidx, iidx, utab, itab):
    mesh = plsc.VectorSubcoreMesh(core_axis_name="c", subcore_axis_name="s")
    k = functools.partial(
        pl.kernel,
        mesh=mesh,
        out_type=[jax.ShapeDtypeStruct((B, D), jnp.float32),
                  jax.ShapeDtypeStruct((B, D), jnp.float32)],
        scratch_types=[
            pltpu.VMEM((CHUNKS, CHUNK), jnp.int32),
            pltpu.VMEM((CHUNKS, CHUNK), jnp.int32),
            pltpu.VMEM((SEG, D), jnp.float32),
            pltpu.VMEM((SEG, D), jnp.float32),
            pltpu.SemaphoreType.DMA,
            pltpu.SemaphoreType.DMA,
            pltpu.SemaphoreType.DMA,
            pltpu.SemaphoreType.DMA,
        ],
    )(_gather_body)
    return k(uidx, iidx, utab, itab)


def _mlp_body(eu, ei, w1a, w1b, b1, w2, b2, wp, bp, out):
    x = (jnp.dot(eu[...], w1a[...], preferred_element_type=jnp.float32)
         + jnp.dot(ei[...], w1b[...], preferred_element_type=jnp.float32)
         + b1[...])
    h = jnp.maximum(x, 0.0)
    h2 = jnp.maximum(
        jnp.dot(h, w2[...], preferred_element_type=jnp.float32) + b2[...], 0.0)
    out[...] = jnp.dot(h2, wp[...], preferred_element_type=jnp.float32) + bp[0, 0]


def _mlp(eu, ei, w1a, w1b, b1, w2, b2, wp, bp):
    BLK = 2048
    grid = (B // BLK,)
    full = lambda i: (0, 0)
    return pl.pallas_call(
        _mlp_body,
        grid=grid,
        in_specs=[
            pl.BlockSpec((BLK, D), lambda i: (i, 0)),
            pl.BlockSpec((BLK, D), lambda i: (i, 0)),
            pl.BlockSpec((D, 64), full),
            pl.BlockSpec((D, 64), full),
            pl.BlockSpec((1, 64), full),
            pl.BlockSpec((64, 16), full),
            pl.BlockSpec((1, 16), full),
            pl.BlockSpec((16, 1), full),
            pl.BlockSpec((1, 1), full),
        ],
        out_specs=pl.BlockSpec((BLK, 1), lambda i: (i, 0)),
        out_shape=jax.ShapeDtypeStruct((B, 1), jnp.float32),
    )(eu, ei, w1a, w1b, b1, w2, b2, wp, bp)


def kernel(user, item, embed_user, embed_item, W1, b1, W2, b2, Wp, bp):
    uidx = user.astype(jnp.int32).reshape(NW, CHUNKS, CHUNK)
    iidx = item.astype(jnp.int32).reshape(NW, CHUNKS, CHUNK)
    eu, ei = _gather(uidx, iidx, embed_user, embed_item)
    w1a = W1[:D]
    w1b = W1[D:]
    pred = _mlp(eu, ei, w1a, w1b,
                b1.reshape(1, 64), W2, b2.reshape(1, 16),
                Wp, bp.reshape(1, 1))
    return pred.reshape(-1)


# speedup vs baseline: 9.6297x; 1.0105x over previous
import functools

import jax
import jax.numpy as jnp
from jax import lax
from jax.experimental import pallas as pl
from jax.experimental.pallas import tpu as pltpu
from jax.experimental.pallas import tpu_sc as plsc

B = 16384
D = 128
NC = 2
NS = 16
NW = NC * NS
PER_W = B // NW
CHUNK = 128
CHUNKS = PER_W // CHUNK


SEG = 256
SEG_CHUNKS = SEG // CHUNK
NSEG = 2 * PER_W // SEG


def _gather_body(uidx_hbm, iidx_hbm, utab_hbm, itab_hbm, out_u, out_i,
                 uidx_v, iidx_v, rows_a, rows_b, gsem_a, gsem_b,
                 osem_a, osem_b):
    wid = lax.axis_index("s") * NC + lax.axis_index("c")
    base = wid * PER_W
    hu = pltpu.async_copy(uidx_hbm.at[pl.ds(base, PER_W)], uidx_v, gsem_a)
    hi = pltpu.async_copy(iidx_hbm.at[pl.ds(base, PER_W)], iidx_v, gsem_b)
    hu.wait()
    hi.wait()
    bufs = [rows_a, rows_b]
    gsems = [gsem_a, gsem_b]
    osems = [osem_a, osem_b]
    segs = [(uidx_v, 0, out_u, base), (uidx_v, SEG_CHUNKS, out_u, base + SEG),
            (iidx_v, 0, out_i, base), (iidx_v, SEG_CHUNKS, out_i, base + SEG)]
    gh = [None] * NSEG
    oh = [None] * NSEG
    for k in range(NSEG + 1):
        if k < NSEG:
            if k >= 2:
                oh[k - 2].wait()
            idxv, coff, _, _ = segs[k]
            b = k % 2
            gh[k] = [pltpu.async_copy(
                (utab_hbm if idxv is uidx_v else itab_hbm)
                .at[idxv.at[pl.ds((coff + j) * CHUNK, CHUNK)]],
                bufs[b].at[pl.ds(j * CHUNK, CHUNK)], gsems[b])
                for j in range(SEG_CHUNKS)]
        if k >= 1:
            p = k - 1
            for h in gh[p]:
                h.wait()
            _, _, outref, roff = segs[p]
            oh[p] = pltpu.async_copy(bufs[p % 2], outref.at[pl.ds(roff, SEG)],
                                     osems[p % 2])
    oh[NSEG - 2].wait()
    oh[NSEG - 1].wait()


@functools.partial(jax.jit, static_argnums=())
def _gather(uidx, iidx, utab, itab):
    mesh = plsc.VectorSubcoreMesh(core_axis_name="c", subcore_axis_name="s")
    k = functools.partial(
        pl.kernel,
        mesh=mesh,
        out_type=[jax.ShapeDtypeStruct((B, D), jnp.float32),
                  jax.ShapeDtypeStruct((B, D), jnp.float32)],
        scratch_types=[
            pltpu.VMEM((PER_W,), jnp.int32),
            pltpu.VMEM((PER_W,), jnp.int32),
            pltpu.VMEM((SEG, D), jnp.float32),
            pltpu.VMEM((SEG, D), jnp.float32),
            pltpu.SemaphoreType.DMA,
            pltpu.SemaphoreType.DMA,
            pltpu.SemaphoreType.DMA,
            pltpu.SemaphoreType.DMA,
        ],
    )(_gather_body)
    return k(uidx, iidx, utab, itab)


def _mlp_body(eu, ei, w1a, w1b, b1, w2, b2, wp, bp, out):
    x = (jnp.dot(eu[...], w1a[...], preferred_element_type=jnp.float32)
         + jnp.dot(ei[...], w1b[...], preferred_element_type=jnp.float32)
         + b1[...])
    h = jnp.maximum(x, 0.0)
    h2 = jnp.maximum(
        jnp.dot(h, w2[...], preferred_element_type=jnp.float32) + b2[...], 0.0)
    out[...] = jnp.dot(h2, wp[...], preferred_element_type=jnp.float32) + bp[0, 0]


def _mlp(eu, ei, w1a, w1b, b1, w2, b2, wp, bp):
    BLK = 2048
    grid = (B // BLK,)
    full = lambda i: (0, 0)
    return pl.pallas_call(
        _mlp_body,
        grid=grid,
        in_specs=[
            pl.BlockSpec((BLK, D), lambda i: (i, 0)),
            pl.BlockSpec((BLK, D), lambda i: (i, 0)),
            pl.BlockSpec((D, 64), full),
            pl.BlockSpec((D, 64), full),
            pl.BlockSpec((1, 64), full),
            pl.BlockSpec((64, 16), full),
            pl.BlockSpec((1, 16), full),
            pl.BlockSpec((16, 1), full),
            pl.BlockSpec((1, 1), full),
        ],
        out_specs=pl.BlockSpec((BLK, 1), lambda i: (i, 0)),
        out_shape=jax.ShapeDtypeStruct((B, 1), jnp.float32),
    )(eu, ei, w1a, w1b, b1, w2, b2, wp, bp)


def kernel(user, item, embed_user, embed_item, W1, b1, W2, b2, Wp, bp):
    eu, ei = _gather(user.astype(jnp.int32), item.astype(jnp.int32),
                     embed_user, embed_item)
    w1a = W1[:D]
    w1b = W1[D:]
    pred = _mlp(eu, ei, w1a, w1b,
                b1.reshape(1, 64), W2, b2.reshape(1, 16),
                Wp, bp.reshape(1, 1))
    return pred.reshape(-1)


# speedup vs baseline: 11.2183x; 1.1650x over previous
import functools

import jax
import jax.numpy as jnp
from jax import lax
from jax.experimental import pallas as pl
from jax.experimental.pallas import tpu as pltpu
from jax.experimental.pallas import tpu_sc as plsc

B = 16384
D = 128
NC = 2
NS = 16
NW = NC * NS
PER_W = B // NW
CHUNK = 128
CHUNKS = PER_W // CHUNK


SEG = 256
SEG_CHUNKS = SEG // CHUNK
NSEG = 2 * PER_W // SEG


def _gather_body(uidx_hbm, iidx_hbm, utab_hbm, itab_hbm, out_u, out_i,
                 uidx_v, iidx_v, rows_a, rows_b, gsem_a, gsem_b,
                 osem_a, osem_b):
    wid = lax.axis_index("s") * NC + lax.axis_index("c")
    base = wid * PER_W
    hu = pltpu.async_copy(uidx_hbm.at[pl.ds(base, PER_W)], uidx_v, gsem_a)
    hi = pltpu.async_copy(iidx_hbm.at[pl.ds(base, PER_W)], iidx_v, gsem_b)
    hu.wait()
    hi.wait()
    bufs = [rows_a, rows_b]
    gsems = [gsem_a, gsem_b]
    osems = [osem_a, osem_b]
    segs = [(uidx_v, 0, out_u, base), (uidx_v, SEG_CHUNKS, out_u, base + SEG),
            (iidx_v, 0, out_i, base), (iidx_v, SEG_CHUNKS, out_i, base + SEG)]
    gh = [None] * NSEG
    oh = [None] * NSEG
    for k in range(NSEG + 1):
        if k < NSEG:
            if k >= 2:
                oh[k - 2].wait()
            idxv, coff, _, _ = segs[k]
            b = k % 2
            gh[k] = [pltpu.async_copy(
                (utab_hbm if idxv is uidx_v else itab_hbm)
                .at[idxv.at[pl.ds((coff + j) * CHUNK, CHUNK)]],
                bufs[b].at[pl.ds(j * CHUNK, CHUNK)], gsems[b])
                for j in range(SEG_CHUNKS)]
        if k >= 1:
            p = k - 1
            for h in gh[p]:
                h.wait()
            _, _, outref, roff = segs[p]
            oh[p] = pltpu.async_copy(bufs[p % 2], outref.at[pl.ds(roff, SEG)],
                                     osems[p % 2])
    oh[NSEG - 2].wait()
    oh[NSEG - 1].wait()


@functools.partial(jax.jit, static_argnums=())
def _gather(uidx, iidx, utab, itab):
    mesh = plsc.VectorSubcoreMesh(core_axis_name="c", subcore_axis_name="s")
    k = functools.partial(
        pl.kernel,
        mesh=mesh,
        out_type=[jax.ShapeDtypeStruct((B, D), jnp.float32),
                  jax.ShapeDtypeStruct((B, D), jnp.float32)],
        scratch_types=[
            pltpu.VMEM((PER_W,), jnp.int32),
            pltpu.VMEM((PER_W,), jnp.int32),
            pltpu.VMEM((SEG, D), jnp.float32),
            pltpu.VMEM((SEG, D), jnp.float32),
            pltpu.SemaphoreType.DMA,
            pltpu.SemaphoreType.DMA,
            pltpu.SemaphoreType.DMA,
            pltpu.SemaphoreType.DMA,
        ],
    )(_gather_body)
    return k(uidx, iidx, utab, itab)


def _mlp_body(eu, ei, w1a, w1b, b1, w2, b2, wp, bp, out):
    x = (jnp.dot(eu[...], w1a[...], preferred_element_type=jnp.float32)
         + jnp.dot(ei[...], w1b[...], preferred_element_type=jnp.float32)
         + b1[...])
    h = jnp.maximum(x, 0.0)
    h2 = jnp.maximum(
        jnp.dot(h, w2[...], preferred_element_type=jnp.float32) + b2[...], 0.0)
    out[...] = (jnp.dot(wp[...], h2.T, preferred_element_type=jnp.float32)
                + bp[0, 0])[None]


def _mlp(eu, ei, w1a, w1b, b1, w2, b2, wp, bp):
    BLK = 2048
    grid = (B // BLK,)
    full = lambda i: (0, 0)
    return pl.pallas_call(
        _mlp_body,
        grid=grid,
        in_specs=[
            pl.BlockSpec((BLK, D), lambda i: (i, 0)),
            pl.BlockSpec((BLK, D), lambda i: (i, 0)),
            pl.BlockSpec((D, 64), full),
            pl.BlockSpec((D, 64), full),
            pl.BlockSpec((1, 64), full),
            pl.BlockSpec((64, 16), full),
            pl.BlockSpec((1, 16), full),
            pl.BlockSpec((1, 16), full),
            pl.BlockSpec((1, 1), full),
        ],
        out_specs=pl.BlockSpec((1, 1, BLK), lambda i: (i, 0, 0)),
        out_shape=jax.ShapeDtypeStruct((B // BLK, 1, BLK), jnp.float32),
        compiler_params=pltpu.CompilerParams(
            dimension_semantics=("parallel",)),
    )(eu, ei, w1a, w1b, b1, w2, b2, wp, bp)


def kernel(user, item, embed_user, embed_item, W1, b1, W2, b2, Wp, bp):
    eu, ei = _gather(user.astype(jnp.int32), item.astype(jnp.int32),
                     embed_user, embed_item)
    w1a = W1[:D]
    w1b = W1[D:]
    pred = _mlp(eu, ei, w1a, w1b,
                b1.reshape(1, 64), W2, b2.reshape(1, 16),
                Wp.reshape(1, 16), bp.reshape(1, 1))
    return pred.reshape(-1)


# speedup vs baseline: 11.6869x; 1.0418x over previous
import functools

import jax
import jax.numpy as jnp
from jax import lax
from jax.experimental import pallas as pl
from jax.experimental.pallas import tpu as pltpu
from jax.experimental.pallas import tpu_sc as plsc

B = 16384
D = 128
NC = 2
NS = 16
NW = NC * NS
PER_W = B // NW
CHUNK = 128
CHUNKS = PER_W // CHUNK


SEG = 256
SEG_CHUNKS = SEG // CHUNK
NSEG = 2 * PER_W // SEG


def _gather_body(uidx_hbm, iidx_hbm, utab_hbm, itab_hbm, out_u, out_i,
                 uidx_v, iidx_v, rows_a, rows_b, gsem_a, gsem_b,
                 osem_a, osem_b):
    wid = lax.axis_index("s") * NC + lax.axis_index("c")
    base = wid * PER_W
    hu = pltpu.async_copy(uidx_hbm.at[pl.ds(base, PER_W)], uidx_v, gsem_a)
    hi = pltpu.async_copy(iidx_hbm.at[pl.ds(base, PER_W)], iidx_v, gsem_b)
    hu.wait()
    hi.wait()
    bufs = [rows_a, rows_b]
    gsems = [gsem_a, gsem_b]
    osems = [osem_a, osem_b]
    segs = [(uidx_v, 0, out_u, base), (uidx_v, SEG_CHUNKS, out_u, base + SEG),
            (iidx_v, 0, out_i, base), (iidx_v, SEG_CHUNKS, out_i, base + SEG)]
    gh = [None] * NSEG
    oh = [None] * NSEG
    for k in range(NSEG + 1):
        if k < NSEG:
            if k >= 2:
                oh[k - 2].wait()
            idxv, coff, _, _ = segs[k]
            b = k % 2
            gh[k] = [pltpu.async_copy(
                (utab_hbm if idxv is uidx_v else itab_hbm)
                .at[idxv.at[pl.ds((coff + j) * CHUNK, CHUNK)]],
                bufs[b].at[pl.ds(j * CHUNK, CHUNK)], gsems[b])
                for j in range(SEG_CHUNKS)]
        if k >= 1:
            p = k - 1
            for h in gh[p]:
                h.wait()
            _, _, outref, roff = segs[p]
            oh[p] = pltpu.async_copy(bufs[p % 2], outref.at[pl.ds(roff, SEG)],
                                     osems[p % 2])
    oh[NSEG - 2].wait()
    oh[NSEG - 1].wait()


@functools.partial(jax.jit, static_argnums=())
def _gather(uidx, iidx, utab, itab):
    mesh = plsc.VectorSubcoreMesh(core_axis_name="c", subcore_axis_name="s")
    k = functools.partial(
        pl.kernel,
        mesh=mesh,
        out_type=[jax.ShapeDtypeStruct((B, D), jnp.float32),
                  jax.ShapeDtypeStruct((B, D), jnp.float32)],
        scratch_types=[
            pltpu.VMEM((PER_W,), jnp.int32),
            pltpu.VMEM((PER_W,), jnp.int32),
            pltpu.VMEM((SEG, D), jnp.float32),
            pltpu.VMEM((SEG, D), jnp.float32),
            pltpu.SemaphoreType.DMA,
            pltpu.SemaphoreType.DMA,
            pltpu.SemaphoreType.DMA,
            pltpu.SemaphoreType.DMA,
        ],
    )(_gather_body)
    return k(uidx, iidx, utab, itab)


def _mlp_body(eu, ei, w1a, w1b, b1, w2, b2, wp, bp, out):
    x = (jnp.dot(eu[...], w1a[...], preferred_element_type=jnp.float32)
         + jnp.dot(ei[...], w1b[...], preferred_element_type=jnp.float32)
         + b1[...])
    h = jnp.maximum(x, 0.0)
    h2 = jnp.maximum(
        jnp.dot(h, w2[...], preferred_element_type=jnp.float32) + b2[...], 0.0)
    out[...] = (jnp.dot(wp[...], h2.T, preferred_element_type=jnp.float32)
                + bp[0, 0])[None]


def _mlp(eu, ei, w1a, w1b, b1, w2, b2, wp, bp):
    BLK = 4096
    grid = (B // BLK,)
    full = lambda i: (0, 0)
    return pl.pallas_call(
        _mlp_body,
        grid=grid,
        in_specs=[
            pl.BlockSpec((BLK, D), lambda i: (i, 0)),
            pl.BlockSpec((BLK, D), lambda i: (i, 0)),
            pl.BlockSpec((D, 64), full),
            pl.BlockSpec((D, 64), full),
            pl.BlockSpec((1, 64), full),
            pl.BlockSpec((64, 16), full),
            pl.BlockSpec((1, 16), full),
            pl.BlockSpec((1, 16), full),
            pl.BlockSpec((1, 1), full),
        ],
        out_specs=pl.BlockSpec((1, 1, BLK), lambda i: (i, 0, 0)),
        out_shape=jax.ShapeDtypeStruct((B // BLK, 1, BLK), jnp.float32),
        compiler_params=pltpu.CompilerParams(
            dimension_semantics=("parallel",)),
    )(eu, ei, w1a, w1b, b1, w2, b2, wp, bp)


def kernel(user, item, embed_user, embed_item, W1, b1, W2, b2, Wp, bp):
    eu, ei = _gather(user.astype(jnp.int32), item.astype(jnp.int32),
                     embed_user, embed_item)
    w1a = W1[:D]
    w1b = W1[D:]
    pred = _mlp(eu, ei, w1a, w1b,
                b1.reshape(1, 64), W2, b2.reshape(1, 16),
                Wp.reshape(1, 16), bp.reshape(1, 1))
    return pred.reshape(-1)
